# baseline (speedup 1.0000x reference)
import functools

import jax
import jax.numpy as jnp
from jax import lax
from jax.experimental import pallas as pl
from jax.experimental.pallas import tpu as pltpu
from jax.experimental.pallas import tpu_sc as plsc

_RB = 128
_JC = 128


def _argmin_body(n_train, x_ref, xt_ref, idx_out_ref, b2_ref):
    i = pl.program_id(0)
    nfull = n_train // _JC
    ntail = n_train - nfull * _JC

    x = x_ref[...]
    a2 = jnp.sum(x * x, axis=1, keepdims=True)

    @pl.when(i == 0)
    def _precompute_b2():
        def bb(c, carry):
            xc = xt_ref[pl.ds(c * _JC, _JC), :]
            b2_ref[0, pl.ds(c * _JC, _JC)] = jnp.sum(xc * xc, axis=1)
            return carry
        lax.fori_loop(0, nfull, bb, 0, unroll=4)
        if ntail:
            xc = xt_ref[pl.ds(nfull * _JC, ntail), :]
            b2_ref[0, pl.ds(nfull * _JC, ntail)] = jnp.sum(xc * xc, axis=1)

    def fold(c, carry):
        run_v, run_b = carry
        xc = xt_ref[pl.ds(c * _JC, _JC), :]
        s = lax.dot_general(
            x, xc, (((1,), (1,)), ((), ())),
            preferred_element_type=jnp.float32,
            precision=lax.Precision.DEFAULT,
        )
        b2 = b2_ref[0, pl.ds(c * _JC, _JC)]
        d2 = (a2 - 2.0 * s) + b2[None, :]
        better = d2 < run_v
        run_v = jnp.where(better, d2, run_v)
        run_b = jnp.where(better, c, run_b)
        return run_v, run_b

    v0 = jnp.full((_RB, _JC), jnp.inf, dtype=jnp.float32)
    b0 = jnp.zeros((_RB, _JC), dtype=jnp.int32)
    run_v, run_b = lax.fori_loop(0, nfull, fold, (v0, b0), unroll=4)

    if ntail:
        xc = xt_ref[pl.ds(nfull * _JC, ntail), :]
        s = lax.dot_general(
            x, xc, (((1,), (1,)), ((), ())),
            preferred_element_type=jnp.float32,
            precision=lax.Precision.DEFAULT,
        )
        b2 = b2_ref[0, pl.ds(nfull * _JC, ntail)]
        d2 = (a2 - 2.0 * s) + b2[None, :]
        pad = jnp.full((_RB, _JC - ntail), jnp.inf, dtype=jnp.float32)
        d2w = jnp.concatenate([d2, pad], axis=1)
        better = d2w < run_v
        run_v = jnp.where(better, d2w, run_v)
        run_b = jnp.where(better, nfull, run_b)

    gmin = jnp.min(run_v, axis=1, keepdims=True)
    lane = lax.broadcasted_iota(jnp.int32, run_v.shape, 1)
    cand = jnp.where(run_v == gmin, run_b * _JC + lane, jnp.int32(2**30))
    idx_out_ref[...] = jnp.min(cand, axis=1, keepdims=True)


def _nearest_idx(x_flat, X_train):
    b, k = x_flat.shape
    n = X_train.shape[0]
    return pl.pallas_call(
        functools.partial(_argmin_body, n),
        grid=(b // _RB,),
        in_specs=[
            pl.BlockSpec((_RB, k), lambda i: (i, 0)),
            pl.BlockSpec((n, k), lambda i: (0, 0)),
        ],
        out_specs=pl.BlockSpec((_RB, 1), lambda i: (i, 0)),
        out_shape=jax.ShapeDtypeStruct((b, 1), jnp.int32),
        scratch_shapes=[
            pltpu.VMEM((1, n), jnp.float32),
        ],
    )(x_flat, X_train)


def _gather_body(bpw, y_hbm, idx_hbm, out_hbm, idx_v, rows_v, sem):
    wid = lax.axis_index("s") * 2 + lax.axis_index("c")
    base = wid * bpw
    pltpu.sync_copy(idx_hbm.at[pl.ds(base, bpw)], idx_v)
    pltpu.async_copy(y_hbm.at[idx_v], rows_v, sem).wait()
    pltpu.sync_copy(rows_v, out_hbm.at[pl.ds(base, bpw)])


def _gather_rows(Y2d, idx):
    b = idx.shape[0]
    d = Y2d.shape[1]
    nw = 32
    bpw = b // nw
    mesh = plsc.VectorSubcoreMesh(core_axis_name="c", subcore_axis_name="s")
    return pl.kernel(
        functools.partial(_gather_body, bpw),
        out_type=jax.ShapeDtypeStruct((b, d), jnp.float32),
        mesh=mesh,
        compiler_params=pltpu.CompilerParams(use_tc_tiling_on_sc=False),
        scratch_types=[
            pltpu.VMEM((bpw,), jnp.int32),
            pltpu.VMEM((bpw, d), jnp.float32),
            pltpu.SemaphoreType.DMA,
        ],
    )(Y2d, idx)


def kernel(x, X_train, Y_train):
    b = x.shape[0]
    x_flat = x.reshape(b, -1)
    idx = _nearest_idx(x_flat, X_train)
    n, dy = Y_train.shape[0], Y_train.shape[1]
    y = _gather_rows(Y_train.reshape(n, dy), idx.reshape(b))
    return y.reshape(b, dy, 1)


# speedup vs baseline: 1.9233x; 1.9233x over previous
import functools

import jax
import jax.numpy as jnp
from jax import lax
from jax.experimental import pallas as pl
from jax.experimental.pallas import tpu as pltpu
from jax.experimental.pallas import tpu_sc as plsc

_JB = 1000


def _fold_body(x_ref, xb_ref, v_out, b_out):
    j = pl.program_id(0)

    @pl.when(j == 0)
    def _init():
        v_out[...] = jnp.full_like(v_out, jnp.inf)
        b_out[...] = jnp.zeros_like(b_out)

    x = x_ref[...]
    xb = xb_ref[...]
    s = lax.dot_general(
        x, xb, (((1,), (1,)), ((), ())),
        preferred_element_type=jnp.float32,
        precision=lax.Precision.DEFAULT,
    )
    a2 = jnp.sum(x * x, axis=1, keepdims=True)
    b2 = jnp.sum(xb * xb, axis=1)
    d2 = (a2 - 2.0 * s) + b2[None, :]

    rv = v_out[...]
    better = d2 < rv
    v_out[...] = jnp.minimum(d2, rv)
    b_out[...] = jnp.where(better, j, b_out[...])


def _extract_body(v_ref, b_ref, idx_ref):
    rv = v_ref[...]
    rb = b_ref[...]
    gmin = jnp.min(rv, axis=1, keepdims=True)
    lane = lax.broadcasted_iota(jnp.int32, rv.shape, 1)
    cand = jnp.where(rv == gmin, rb * _JB + lane, jnp.int32(2**30))
    idx_ref[...] = jnp.min(cand, axis=1, keepdims=True)


def _nearest_idx(x_flat, X_train):
    b, k = x_flat.shape
    n = X_train.shape[0]
    nj = n // _JB
    run_v, run_b = pl.pallas_call(
        _fold_body,
        grid=(nj,),
        in_specs=[
            pl.BlockSpec((b, k), lambda j: (0, 0)),
            pl.BlockSpec((_JB, k), lambda j: (j, 0)),
        ],
        out_specs=[
            pl.BlockSpec((b, _JB), lambda j: (0, 0)),
            pl.BlockSpec((b, _JB), lambda j: (0, 0)),
        ],
        out_shape=[
            jax.ShapeDtypeStruct((b, _JB), jnp.float32),
            jax.ShapeDtypeStruct((b, _JB), jnp.int32),
        ],
    )(x_flat, X_train)

    rb_rows = 128
    return pl.pallas_call(
        _extract_body,
        grid=(b // rb_rows,),
        in_specs=[
            pl.BlockSpec((rb_rows, _JB), lambda i: (i, 0)),
            pl.BlockSpec((rb_rows, _JB), lambda i: (i, 0)),
        ],
        out_specs=pl.BlockSpec((rb_rows, 1), lambda i: (i, 0)),
        out_shape=jax.ShapeDtypeStruct((b, 1), jnp.int32),
    )(run_v, run_b)


def _gather_body(bpw, y_hbm, idx_hbm, out_hbm, idx_v, rows_v, sem):
    wid = lax.axis_index("s") * 2 + lax.axis_index("c")
    base = wid * bpw
    pltpu.sync_copy(idx_hbm.at[pl.ds(base, bpw)], idx_v)
    pltpu.async_copy(y_hbm.at[idx_v], rows_v, sem).wait()
    pltpu.sync_copy(rows_v, out_hbm.at[pl.ds(base, bpw)])


def _gather_rows(Y2d, idx):
    b = idx.shape[0]
    d = Y2d.shape[1]
    nw = 32
    bpw = b // nw
    mesh = plsc.VectorSubcoreMesh(core_axis_name="c", subcore_axis_name="s")
    return pl.kernel(
        functools.partial(_gather_body, bpw),
        out_type=jax.ShapeDtypeStruct((b, d), jnp.float32),
        mesh=mesh,
        compiler_params=pltpu.CompilerParams(use_tc_tiling_on_sc=False),
        scratch_types=[
            pltpu.VMEM((bpw,), jnp.int32),
            pltpu.VMEM((bpw, d), jnp.float32),
            pltpu.SemaphoreType.DMA,
        ],
    )(Y2d, idx)


def kernel(x, X_train, Y_train):
    b = x.shape[0]
    x_flat = x.reshape(b, -1)
    idx = _nearest_idx(x_flat, X_train)
    n, dy = Y_train.shape[0], Y_train.shape[1]
    y = _gather_rows(Y_train.reshape(n, dy), idx.reshape(b))
    return y.reshape(b, dy, 1)


# speedup vs baseline: 2.4454x; 1.2714x over previous
import functools

import jax
import jax.numpy as jnp
from jax import lax
from jax.experimental import pallas as pl
from jax.experimental.pallas import tpu as pltpu
from jax.experimental.pallas import tpu_sc as plsc

_JB = 1000


def _fold_body(x_ref, xtb_ref, v_out, b_out):
    j = pl.program_id(0)

    @pl.when(j == 0)
    def _init():
        v_out[...] = jnp.full_like(v_out, jnp.inf)
        b_out[...] = jnp.zeros_like(b_out)

    x = x_ref[...]
    xtb = xtb_ref[0]
    s = lax.dot_general(
        x, xtb, (((1,), (0,)), ((), ())),
        preferred_element_type=jnp.float32,
        precision=lax.Precision.DEFAULT,
    )
    a2 = jnp.sum(x * x, axis=1, keepdims=True)
    b2 = jnp.sum(xtb * xtb, axis=0)
    d2 = (a2 - 2.0 * s) + b2[None, :]

    rv = v_out[...]
    better = d2 < rv
    v_out[...] = jnp.where(better, d2, rv)
    b_out[...] = jnp.where(better, j, b_out[...])


def _extract_body(v_ref, b_ref, idx_ref):
    rv = v_ref[...]
    rb = b_ref[...]
    gmin = jnp.min(rv, axis=1, keepdims=True)
    lane = lax.broadcasted_iota(jnp.int32, rv.shape, 1)
    cand = jnp.where(rv == gmin, rb * _JB + lane, jnp.int32(2**30))
    idx_ref[...] = jnp.min(cand, axis=1, keepdims=True)


def _nearest_idx(x_flat, X_train):
    b, k = x_flat.shape
    n = X_train.shape[0]
    nj = n // _JB
    xt3 = X_train.T.reshape(k, nj, _JB).transpose(1, 0, 2)
    run_v, run_b = pl.pallas_call(
        _fold_body,
        grid=(nj,),
        in_specs=[
            pl.BlockSpec((b, k), lambda j: (0, 0)),
            pl.BlockSpec((1, k, _JB), lambda j: (j, 0, 0)),
        ],
        out_specs=[
            pl.BlockSpec((b, _JB), lambda j: (0, 0)),
            pl.BlockSpec((b, _JB), lambda j: (0, 0)),
        ],
        out_shape=[
            jax.ShapeDtypeStruct((b, _JB), jnp.float32),
            jax.ShapeDtypeStruct((b, _JB), jnp.int32),
        ],
    )(x_flat, xt3)

    rb_rows = 128
    return pl.pallas_call(
        _extract_body,
        grid=(b // rb_rows,),
        in_specs=[
            pl.BlockSpec((rb_rows, _JB), lambda i: (i, 0)),
            pl.BlockSpec((rb_rows, _JB), lambda i: (i, 0)),
        ],
        out_specs=pl.BlockSpec((rb_rows, 1), lambda i: (i, 0)),
        out_shape=jax.ShapeDtypeStruct((b, 1), jnp.int32),
    )(run_v, run_b)


def _gather_body(bpw, y_hbm, idx_hbm, out_hbm, idx_v, rows_v, sem):
    wid = lax.axis_index("s") * 2 + lax.axis_index("c")
    base = wid * bpw
    pltpu.sync_copy(idx_hbm.at[pl.ds(base, bpw)], idx_v)
    pltpu.async_copy(y_hbm.at[idx_v], rows_v, sem).wait()
    pltpu.sync_copy(rows_v, out_hbm.at[pl.ds(base, bpw)])


def _gather_rows(Y2d, idx):
    b = idx.shape[0]
    d = Y2d.shape[1]
    nw = 32
    bpw = b // nw
    mesh = plsc.VectorSubcoreMesh(core_axis_name="c", subcore_axis_name="s")
    return pl.kernel(
        functools.partial(_gather_body, bpw),
        out_type=jax.ShapeDtypeStruct((b, d), jnp.float32),
        mesh=mesh,
        compiler_params=pltpu.CompilerParams(use_tc_tiling_on_sc=False),
        scratch_types=[
            pltpu.VMEM((bpw,), jnp.int32),
            pltpu.VMEM((bpw, d), jnp.float32),
            pltpu.SemaphoreType.DMA,
        ],
    )(Y2d, idx)


def kernel(x, X_train, Y_train):
    b = x.shape[0]
    x_flat = x.reshape(b, -1)
    idx = _nearest_idx(x_flat, X_train)
    n, dy = Y_train.shape[0], Y_train.shape[1]
    y = _gather_rows(Y_train.reshape(n, dy), idx.reshape(b))
    return y.reshape(b, dy, 1)
